# Initial kernel scaffold; baseline (speedup 1.0000x reference)
import jax
import jax.numpy as jnp
from jax.experimental import pallas as pl


def kernel(feature, edge_index, W, b):
    raise NotImplementedError("write your pallas kernel here")


# speedup vs baseline: 5.9844x; 5.9844x over previous
import dataclasses
import functools

import jax
import jax.numpy as jnp
from jax import lax
from jax.experimental import pallas as pl
from jax.experimental.pallas import tpu as pltpu
from jax.experimental.pallas import tpu_sc as plsc

N = 10000
E = 320000
D = 128
NSC = 2
NSUB = 16
NW = NSC * NSUB
EPW = E // NW
CH = 80
NCH = EPW // CH
NP = 10240
STRIPE = NP // NSUB


def _sc_aggregate(feature, ei_flat, zrows):
    mesh = plsc.VectorSubcoreMesh(core_axis_name="c", subcore_axis_name="s")
    cp = pltpu.CompilerParams()
    if "needs_layout_passes" in pltpu.CompilerParams.__dataclass_fields__:
        cp = dataclasses.replace(cp, needs_layout_passes=False)

    @functools.partial(
        pl.kernel,
        mesh=mesh,
        compiler_params=cp,
        out_type=(jax.ShapeDtypeStruct((NSC, NP, D), jnp.float32),
                  jax.ShapeDtypeStruct((NW, NP), jnp.float32)),
        scratch_types=[
            pltpu.VMEM_SHARED((NP, D), jnp.float32),
            pltpu.VMEM((1, CH), jnp.int32),
            pltpu.VMEM((1, CH), jnp.int32),
            pltpu.VMEM((CH, D), jnp.float32),
            pltpu.VMEM((NP,), jnp.float32),
        ],
    )
    def k(f_hbm, ei_hbm, z_hbm, sums_hbm, cnt_hbm, acc_sh, src_v, dst_v,
          rows_v, hist_v):
        cid = lax.axis_index("c")
        sid = lax.axis_index("s")
        base = (cid * NSUB + sid) * EPW

        pltpu.sync_copy(z_hbm, acc_sh.at[pl.ds(sid * STRIPE, STRIPE)])

        @pl.loop(0, NP, step=16)
        def _(i):
            hist_v[pl.ds(i, 16)] = jnp.zeros((16,), jnp.float32)

        plsc.subcore_barrier()
        ones16 = jnp.ones((16,), jnp.float32)

        @pl.loop(0, NCH)
        def _(i):
            off = base + i * CH
            pltpu.sync_copy(ei_hbm.at[pl.ds(off, CH)], src_v.at[0])
            pltpu.sync_copy(ei_hbm.at[pl.ds(E + off, CH)], dst_v.at[0])
            pltpu.sync_copy(f_hbm.at[src_v.at[0]], rows_v)
            pltpu.sync_copy(rows_v, acc_sh.at[dst_v.at[0]], add=True)
            for j in range(CH // 16):
                iv = dst_v[0, pl.ds(j * 16, 16)]
                plsc.addupdate_scatter(hist_v, [iv], ones16)

        pltpu.sync_copy(hist_v, cnt_hbm.at[cid * NSUB + sid])
        plsc.subcore_barrier()
        pltpu.sync_copy(acc_sh.at[pl.ds(sid * STRIPE, STRIPE)],
                        sums_hbm.at[cid, pl.ds(sid * STRIPE, STRIPE)])

    return k(feature, ei_flat, zrows)


def _tc_epilogue(acc, cnt, feature, W, b2):
    def body(acc_ref, c_ref, f_ref, w_ref, b_ref, o_ref):
        sums = acc_ref[0, :N, :] + acc_ref[1, :N, :]
        agg = sums / jnp.maximum(c_ref[...], 1.0)
        h = (jnp.dot(agg, w_ref[:D, :], preferred_element_type=jnp.float32)
             + jnp.dot(f_ref[...], w_ref[D:, :], preferred_element_type=jnp.float32)
             + b_ref[...])
        nrm2 = jnp.sum(h * h, axis=1, keepdims=True)
        o_ref[...] = h * lax.rsqrt(jnp.maximum(nrm2, 1e-24))

    return pl.pallas_call(
        body,
        out_shape=jax.ShapeDtypeStruct((N, D), jnp.float32),
    )(acc, cnt, feature, W, b2)


def kernel(feature, edge_index, W, b):
    zrows = jnp.zeros((STRIPE, D), jnp.float32)
    acc, cparts = _sc_aggregate(feature, edge_index.reshape(-1), zrows)
    cnt = cparts.sum(axis=0)[:N, None]
    return _tc_epilogue(acc, cnt, feature, W, b.reshape(1, D))
